# baseline (speedup 1.0000x reference)
import math

import jax
import jax.numpy as jnp
from jax.experimental import pallas as pl
from jax.experimental.pallas import tpu as pltpu

S_BLK = 256


D_CHUNK = 256


def _pe_kernel(ts_ref, x_ref, pe_ref, ht_ref, dt_ref, w_ref, b_ref, o_ref,
               p0_ref, q0_ref):
    ts = ts_ref[...]
    hour = (ts // 3600) % 24
    day = (ts // 86400 + 3) % 7

    hw = jnp.dot(ht_ref[...], w_ref[0:256, :], preferred_element_type=jnp.float32)
    hw = hw + b_ref[...]
    dw = jnp.dot(dt_ref[...], w_ref[256:512, :], preferred_element_type=jnp.float32)
    cat = jnp.concatenate([hw, dw], axis=0)

    nb = ts.shape[0]
    s_blk = ts.shape[1]
    d = o_ref.shape[2]

    jrow = jax.lax.broadcasted_iota(jnp.int32, (1, d), 1)
    even = (jrow & 1) == 0

    @pl.when(pl.program_id(0) == 0)
    def _init_base():
        p0 = pe_ref[:, 0, :]
        left = jnp.roll(p0, -1, axis=1)
        right = jnp.roll(p0, 1, axis=1)
        p0_ref[...] = p0
        q0_ref[...] = jnp.where(even, left, right)

    freq = jnp.exp((jrow & ~1).astype(jnp.float32) * (-math.log(10000.0) / d))
    ang = (pl.program_id(0) * s_blk).astype(jnp.float32) * freq
    c_row = jnp.cos(ang)
    s_raw = jnp.sin(ang)
    s_row = jnp.where(even, s_raw, -s_raw)

    iota_c = jax.lax.broadcasted_iota(jnp.int32, (31, s_blk), 0)
    dims = (((0,), (0,)), ((), ()))
    pe_blk = p0_ref[...] * c_row + q0_ref[...] * s_row
    for bb in range(nb):
        hit = (hour[bb:bb + 1, :] == iota_c) | (day[bb:bb + 1, :] + 24 == iota_c)
        oh = hit.astype(jnp.float32)
        temporal = jax.lax.dot_general(
            oh, cat, dims, preferred_element_type=jnp.float32)
        o_ref[:, bb, :] = x_ref[:, bb, :] + pe_blk + temporal


def kernel(x, hour_table, day_table, W, b, pe, timestamps):
    S, B, D = x.shape
    b2 = b.reshape(1, D)

    grid = (S // S_BLK,)
    return pl.pallas_call(
        _pe_kernel,
        grid=grid,
        in_specs=[
            pl.BlockSpec((B, S_BLK), lambda i: (0, i)),
            pl.BlockSpec((S_BLK, B, D), lambda i: (i, 0, 0)),
            pl.BlockSpec((S_BLK, 1, D), lambda i: (0, 0, 0)),
            pl.BlockSpec(hour_table.shape, lambda i: (0, 0)),
            pl.BlockSpec(day_table.shape, lambda i: (0, 0)),
            pl.BlockSpec(W.shape, lambda i: (0, 0)),
            pl.BlockSpec((1, D), lambda i: (0, 0)),
        ],
        out_specs=pl.BlockSpec((S_BLK, B, D), lambda i: (i, 0, 0)),
        out_shape=jax.ShapeDtypeStruct((S, B, D), jnp.float32),
        scratch_shapes=[pltpu.VMEM((S_BLK, D), jnp.float32),
                        pltpu.VMEM((S_BLK, D), jnp.float32)],
        compiler_params=pltpu.CompilerParams(
            dimension_semantics=("arbitrary",)),
    )(timestamps, x, pe[:S_BLK], hour_table, day_table, W, b2)


# speedup vs baseline: 1.0462x; 1.0462x over previous
import math

import jax
import jax.numpy as jnp
from jax.experimental import pallas as pl
from jax.experimental.pallas import tpu as pltpu

S_BLK = 512


D_CHUNK = 256


def _pe_kernel(ts_ref, x_ref, pe_ref, ht_ref, dt_ref, w_ref, b_ref, o_ref,
               p0_ref, q0_ref):
    ts = ts_ref[...]
    hour = (ts // 3600) % 24
    day = (ts // 86400 + 3) % 7

    hw = jnp.dot(ht_ref[...], w_ref[0:256, :], preferred_element_type=jnp.float32)
    hw = hw + b_ref[...]
    dw = jnp.dot(dt_ref[...], w_ref[256:512, :], preferred_element_type=jnp.float32)
    cat = jnp.concatenate([hw, dw], axis=0)

    nb = ts.shape[0]
    s_blk = ts.shape[1]
    d = o_ref.shape[2]

    jrow = jax.lax.broadcasted_iota(jnp.int32, (1, d), 1)
    even = (jrow & 1) == 0

    @pl.when(pl.program_id(0) == 0)
    def _init_base():
        p0 = pe_ref[:, 0, :]
        left = jnp.roll(p0, -1, axis=1)
        right = jnp.roll(p0, 1, axis=1)
        p0_ref[...] = p0
        q0_ref[...] = jnp.where(even, left, right)

    freq = jnp.exp((jrow & ~1).astype(jnp.float32) * (-math.log(10000.0) / d))
    ang = (pl.program_id(0) * s_blk).astype(jnp.float32) * freq
    c_row = jnp.cos(ang)
    s_raw = jnp.sin(ang)
    s_row = jnp.where(even, s_raw, -s_raw)

    iota_c = jax.lax.broadcasted_iota(jnp.int32, (31, s_blk), 0)
    dims = (((0,), (0,)), ((), ()))
    pe_blk = p0_ref[...] * c_row + q0_ref[...] * s_row
    for bb in range(nb):
        hit = (hour[bb:bb + 1, :] == iota_c) | (day[bb:bb + 1, :] + 24 == iota_c)
        oh = hit.astype(jnp.float32)
        temporal = jax.lax.dot_general(
            oh, cat, dims, preferred_element_type=jnp.float32)
        o_ref[:, bb, :] = x_ref[:, bb, :] + pe_blk + temporal


def kernel(x, hour_table, day_table, W, b, pe, timestamps):
    S, B, D = x.shape
    b2 = b.reshape(1, D)

    grid = (S // S_BLK,)
    return pl.pallas_call(
        _pe_kernel,
        grid=grid,
        in_specs=[
            pl.BlockSpec((B, S_BLK), lambda i: (0, i)),
            pl.BlockSpec((S_BLK, B, D), lambda i: (i, 0, 0)),
            pl.BlockSpec((S_BLK, 1, D), lambda i: (0, 0, 0)),
            pl.BlockSpec(hour_table.shape, lambda i: (0, 0)),
            pl.BlockSpec(day_table.shape, lambda i: (0, 0)),
            pl.BlockSpec(W.shape, lambda i: (0, 0)),
            pl.BlockSpec((1, D), lambda i: (0, 0)),
        ],
        out_specs=pl.BlockSpec((S_BLK, B, D), lambda i: (i, 0, 0)),
        out_shape=jax.ShapeDtypeStruct((S, B, D), jnp.float32),
        scratch_shapes=[pltpu.VMEM((S_BLK, D), jnp.float32),
                        pltpu.VMEM((S_BLK, D), jnp.float32)],
        compiler_params=pltpu.CompilerParams(
            dimension_semantics=("arbitrary",)),
    )(timestamps, x, pe[:S_BLK], hour_table, day_table, W, b2)
